# Initial kernel scaffold; baseline (speedup 1.0000x reference)
import jax
import jax.numpy as jnp
from jax.experimental import pallas as pl


def kernel(story, C0, C1, C2, C3):
    raise NotImplementedError("write your pallas kernel here")


# speedup vs baseline: 10.3436x; 10.3436x over previous
import functools

import jax
import jax.numpy as jnp
from jax import lax
from jax.experimental import pallas as pl
from jax.experimental.pallas import tpu as pltpu
from jax.experimental.pallas import tpu_sc as plsc

DIM = 64
T = 4
NC, NS = 2, 16
NW = NC * NS
W = 128


def _sc_gather_sum(story_r3, c1, c2, c3, iota):
    n_win = story_r3.shape[0]
    S = n_win * W
    wins_per_sub = n_win // NW
    mesh = plsc.VectorSubcoreMesh(core_axis_name="c", subcore_axis_name="s")
    out_t = tuple(jax.ShapeDtypeStruct((S, DIM), jnp.float32) for _ in range(3))

    @functools.partial(
        pl.kernel,
        mesh=mesh,
        out_type=out_t,
        compiler_params=pltpu.CompilerParams(use_tc_tiling_on_sc=False),
        scratch_types=[
            pltpu.VMEM((T, W), jnp.int32),
            pltpu.VMEM((T, W, DIM), jnp.float32),
            pltpu.VMEM_SHARED((NS * W, DIM), jnp.float32),
            pltpu.VMEM((W,), jnp.int32),
            pltpu.SemaphoreType.DMA,
        ],
    )
    def k(story_hbm, c1_hbm, c2_hbm, c3_hbm, iota_hbm,
          m1_hbm, m2_hbm, m3_hbm, idx_v, rows_v, acc_sh, iota_v, sem):
        sid = lax.axis_index("s")
        wid = sid * NC + lax.axis_index("c")
        pltpu.sync_copy(iota_hbm.at[sid], iota_v)
        for tbl, out in ((c1_hbm, m1_hbm), (c2_hbm, m2_hbm), (c3_hbm, m3_hbm)):
            @pl.loop(0, wins_per_sub)
            def _(w, tbl=tbl, out=out):
                win = wid * wins_per_sub + w
                pltpu.sync_copy(story_hbm.at[win], idx_v)
                cps = [pltpu.async_copy(tbl.at[idx_v.at[t]], rows_v.at[t], sem)
                       for t in range(T)]
                for cp in cps:
                    cp.wait()
                pltpu.sync_copy(rows_v.at[0], acc_sh.at[pl.ds(sid * W, W)])
                for t in range(1, T):
                    pltpu.sync_copy(rows_v.at[t], acc_sh.at[iota_v], add=True)
                pltpu.sync_copy(acc_sh.at[pl.ds(sid * W, W)],
                                out.at[pl.ds(win * W, W)])

    return k(story_r3, c1, c2, c3, iota)


def _attention(m1, m2, m3):
    B, M, _ = m1.shape
    BBLK = 64

    def body(m1_ref, m2_ref, m3_ref, out_ref):
        m1v = m1_ref[...]
        u1 = jnp.mean(m1v, axis=1)
        l1 = jnp.sum(m1v * u1[:, None, :], axis=2)
        p1 = jax.nn.softmax(l1, axis=1)
        m2v = m2_ref[...]
        u2 = u1 + jnp.sum(m2v * p1[:, :, None], axis=1)
        l2 = jnp.sum(m2v * u2[:, None, :], axis=2)
        p2 = jax.nn.softmax(l2, axis=1)
        m3v = m3_ref[...]
        u3 = u2 + jnp.sum(m3v * p2[:, :, None], axis=1)
        out_ref[...] = u3

    return pl.pallas_call(
        body,
        grid=(B // BBLK,),
        in_specs=[pl.BlockSpec((BBLK, M, DIM), lambda i: (i, 0, 0))] * 3,
        out_specs=pl.BlockSpec((BBLK, DIM), lambda i: (i, 0)),
        out_shape=jax.ShapeDtypeStruct((B, DIM), jnp.float32),
    )(m1, m2, m3)


def kernel(story, C0, C1, C2, C3):
    M, B, T_ = story.shape
    S = B * M
    segs = story.transpose(1, 0, 2).reshape(S, T_)
    story_r3 = segs.reshape(S // W, W, T_).transpose(0, 2, 1)
    iota = (jnp.arange(NS, dtype=jnp.int32)[:, None] * W
            + jnp.arange(W, dtype=jnp.int32)[None, :])
    m1, m2, m3 = _sc_gather_sum(story_r3, C1, C2, C3, iota)
    return _attention(m1.reshape(B, M, DIM),
                      m2.reshape(B, M, DIM),
                      m3.reshape(B, M, DIM))


# speedup vs baseline: 11.8929x; 1.1498x over previous
import functools

import jax
import jax.numpy as jnp
from jax import lax
from jax.experimental import pallas as pl
from jax.experimental.pallas import tpu as pltpu
from jax.experimental.pallas import tpu_sc as plsc

DIM = 64
T = 4
NC, NS = 2, 16
NW = NC * NS
W = 128


def _sc_gather_sum(story_r3, c1, c2, c3, iota):
    n_win = story_r3.shape[0]
    S = n_win * W
    wins_per_sub = n_win // NW
    mesh = plsc.VectorSubcoreMesh(core_axis_name="c", subcore_axis_name="s")
    out_t = tuple(jax.ShapeDtypeStruct((S, DIM), jnp.float32) for _ in range(3))

    @functools.partial(
        pl.kernel,
        mesh=mesh,
        out_type=out_t,
        compiler_params=pltpu.CompilerParams(use_tc_tiling_on_sc=False),
        scratch_types=[
            pltpu.VMEM((2, T, W), jnp.int32),
            pltpu.VMEM((2, T, W, DIM), jnp.float32),
            pltpu.VMEM_SHARED((NS * 2 * W, DIM), jnp.float32),
            pltpu.VMEM((2, W), jnp.int32),
            pltpu.SemaphoreType.DMA,
            pltpu.SemaphoreType.DMA,
            pltpu.SemaphoreType.DMA,
            pltpu.SemaphoreType.DMA,
        ],
    )
    def k(story_hbm, c1_hbm, c2_hbm, c3_hbm, iota_hbm,
          m1_hbm, m2_hbm, m3_hbm, idx_v, rows_v, acc_sh, iota_v,
          sem_i, sem_g, sem_o0, sem_o1):
        sem_o = (sem_o0, sem_o1)
        sid = lax.axis_index("s")
        wid = sid * NC + lax.axis_index("c")
        base = wid * wins_per_sub
        pltpu.sync_copy(iota_hbm.at[sid], iota_v)

        def acc_slice(p):
            return acc_sh.at[pl.ds((sid * 2 + p) * W, W)]

        def issue_idx(w, p):
            pltpu.async_copy(story_hbm.at[base + w], idx_v.at[p], sem_i)

        def wait_idx(p):
            pltpu.make_async_copy(story_hbm.at[base], idx_v.at[p], sem_i).wait()

        def issue_gathers(tbl, p):
            for t in range(T):
                pltpu.async_copy(tbl.at[idx_v.at[p, t]], rows_v.at[p, t], sem_g)

        def wait_gathers(tbl, p):
            for t in range(T):
                pltpu.make_async_copy(tbl.at[idx_v.at[p, t]], rows_v.at[p, t],
                                      sem_g).wait()

        def reduce_and_out(out, w, p):
            pltpu.sync_copy(rows_v.at[p, 0], acc_slice(p))
            for t in range(1, T):
                pltpu.sync_copy(rows_v.at[p, t], acc_sh.at[iota_v.at[p]],
                                add=True)
            pltpu.async_copy(acc_slice(p), out.at[pl.ds((base + w) * W, W)],
                             sem_o[p])

        def wait_out(out, p):
            pltpu.make_async_copy(acc_slice(p), out.at[pl.ds(base * W, W)],
                                  sem_o[p]).wait()

        for tbl, out in ((c1_hbm, m1_hbm), (c2_hbm, m2_hbm), (c3_hbm, m3_hbm)):
            pltpu.sync_copy(story_hbm.at[base], idx_v.at[0])
            issue_gathers(tbl, 0)
            issue_idx(1, 1)

            @pl.loop(0, wins_per_sub // 2)
            def _(i, tbl=tbl, out=out):
                for p in range(2):
                    w = 2 * i + p
                    wait_gathers(tbl, p)
                    q = 1 - p

                    @pl.when(w >= 2)
                    def _():
                        wait_out(out, p)
                    reduce_and_out(out, w, p)

                    @pl.when(w + 1 <= wins_per_sub - 1)
                    def _():
                        wait_idx(q)
                        issue_gathers(tbl, q)

                    @pl.when(w + 2 <= wins_per_sub - 1)
                    def _():
                        issue_idx(w + 2, p)
            wait_out(out, 0)
            wait_out(out, 1)

    return k(story_r3, c1, c2, c3, iota)


def _attention(m1, m2, m3):
    B, M, _ = m1.shape
    BBLK = 64

    def body(m1_ref, m2_ref, m3_ref, out_ref):
        m1v = m1_ref[...]
        u1 = jnp.mean(m1v, axis=1)
        l1 = jnp.sum(m1v * u1[:, None, :], axis=2)
        p1 = jax.nn.softmax(l1, axis=1)
        m2v = m2_ref[...]
        u2 = u1 + jnp.sum(m2v * p1[:, :, None], axis=1)
        l2 = jnp.sum(m2v * u2[:, None, :], axis=2)
        p2 = jax.nn.softmax(l2, axis=1)
        m3v = m3_ref[...]
        u3 = u2 + jnp.sum(m3v * p2[:, :, None], axis=1)
        out_ref[...] = u3

    return pl.pallas_call(
        body,
        grid=(B // BBLK,),
        in_specs=[pl.BlockSpec((BBLK, M, DIM), lambda i: (i, 0, 0))] * 3,
        out_specs=pl.BlockSpec((BBLK, DIM), lambda i: (i, 0)),
        out_shape=jax.ShapeDtypeStruct((B, DIM), jnp.float32),
    )(m1, m2, m3)


def kernel(story, C0, C1, C2, C3):
    M, B, T_ = story.shape
    S = B * M
    segs = story.transpose(1, 0, 2).reshape(S, T_)
    story_r3 = segs.reshape(S // W, W, T_).transpose(0, 2, 1)
    iota = ((jnp.arange(NS, dtype=jnp.int32)[:, None] * 2
             + jnp.arange(2, dtype=jnp.int32)[None, :])[:, :, None] * W
            + jnp.arange(W, dtype=jnp.int32)[None, None, :])
    m1, m2, m3 = _sc_gather_sum(story_r3, C1, C2, C3, iota)
    return _attention(m1.reshape(B, M, DIM),
                      m2.reshape(B, M, DIM),
                      m3.reshape(B, M, DIM))


# speedup vs baseline: 14.4455x; 1.2146x over previous
import functools

import jax
import jax.numpy as jnp
from jax import lax
from jax.experimental import pallas as pl
from jax.experimental.pallas import tpu as pltpu
from jax.experimental.pallas import tpu_sc as plsc

DIM = 64
T = 4
NC, NS = 2, 16
NW = NC * NS
W = 128


def _sc_gather_sum(story_r3, c1, c2, c3):
    n_win = story_r3.shape[0]
    S = n_win * W
    wins_per_sub = n_win // NW
    mesh = plsc.VectorSubcoreMesh(core_axis_name="c", subcore_axis_name="s")
    out_t = tuple(jax.ShapeDtypeStruct((S, DIM), jnp.float32) for _ in range(3))

    @functools.partial(
        pl.kernel,
        mesh=mesh,
        out_type=out_t,
        compiler_params=pltpu.CompilerParams(use_tc_tiling_on_sc=False),
        scratch_types=[
            pltpu.VMEM((2, T, W), jnp.int32),
            pltpu.VMEM((2, T, W, DIM), jnp.float32),
            pltpu.VMEM((2, W, DIM), jnp.float32),
            pltpu.SemaphoreType.DMA,
            pltpu.SemaphoreType.DMA,
            pltpu.SemaphoreType.DMA,
            pltpu.SemaphoreType.DMA,
        ],
    )
    def k(story_hbm, c1_hbm, c2_hbm, c3_hbm,
          m1_hbm, m2_hbm, m3_hbm, idx_v, rows_v, acc_v,
          sem_i, sem_g, sem_o0, sem_o1):
        sem_o = (sem_o0, sem_o1)
        wid = lax.axis_index("s") * NC + lax.axis_index("c")
        base = wid * wins_per_sub

        def acc_slice(p):
            return acc_v.at[p]

        def issue_idx(w, p):
            pltpu.async_copy(story_hbm.at[base + w], idx_v.at[p], sem_i)

        def wait_idx(p):
            pltpu.make_async_copy(story_hbm.at[base], idx_v.at[p], sem_i).wait()

        def issue_gathers(tbl, p):
            for t in range(T):
                pltpu.async_copy(tbl.at[idx_v.at[p, t]], rows_v.at[p, t], sem_g)

        def wait_gathers(tbl, p):
            for t in range(T):
                pltpu.make_async_copy(tbl.at[idx_v.at[p, t]], rows_v.at[p, t],
                                      sem_g).wait()

        def reduce_and_out(out, w, p):
            @pl.loop(0, W)
            def _(j):
                for g in range(DIM // 16):
                    sl = pl.ds(g * 16, 16)
                    acc_v[p, j, sl] = (
                        (rows_v[p, 0, j, sl] + rows_v[p, 1, j, sl])
                        + (rows_v[p, 2, j, sl] + rows_v[p, 3, j, sl]))
            pltpu.async_copy(acc_slice(p), out.at[pl.ds((base + w) * W, W)],
                             sem_o[p])

        def wait_out(out, p):
            pltpu.make_async_copy(acc_slice(p), out.at[pl.ds(base * W, W)],
                                  sem_o[p]).wait()

        for tbl, out in ((c1_hbm, m1_hbm), (c2_hbm, m2_hbm), (c3_hbm, m3_hbm)):
            pltpu.sync_copy(story_hbm.at[base], idx_v.at[0])
            issue_gathers(tbl, 0)
            issue_idx(1, 1)

            @pl.loop(0, wins_per_sub // 2)
            def _(i, tbl=tbl, out=out):
                for p in range(2):
                    w = 2 * i + p
                    wait_gathers(tbl, p)
                    q = 1 - p

                    @pl.when(w + 1 <= wins_per_sub - 1)
                    def _():
                        wait_idx(q)
                        issue_gathers(tbl, q)

                    @pl.when(w + 2 <= wins_per_sub - 1)
                    def _():
                        issue_idx(w + 2, p)

                    @pl.when(w >= 2)
                    def _():
                        wait_out(out, p)
                    reduce_and_out(out, w, p)
            wait_out(out, 0)
            wait_out(out, 1)

    return k(story_r3, c1, c2, c3)


def _attention(m1, m2, m3):
    B, M, _ = m1.shape
    BBLK = 64

    def body(m1_ref, m2_ref, m3_ref, out_ref):
        m1v = m1_ref[...]
        u1 = jnp.mean(m1v, axis=1)
        l1 = jnp.sum(m1v * u1[:, None, :], axis=2)
        p1 = jax.nn.softmax(l1, axis=1)
        m2v = m2_ref[...]
        u2 = u1 + jnp.sum(m2v * p1[:, :, None], axis=1)
        l2 = jnp.sum(m2v * u2[:, None, :], axis=2)
        p2 = jax.nn.softmax(l2, axis=1)
        m3v = m3_ref[...]
        u3 = u2 + jnp.sum(m3v * p2[:, :, None], axis=1)
        out_ref[...] = u3

    return pl.pallas_call(
        body,
        grid=(B // BBLK,),
        in_specs=[pl.BlockSpec((BBLK, M, DIM), lambda i: (i, 0, 0))] * 3,
        out_specs=pl.BlockSpec((BBLK, DIM), lambda i: (i, 0)),
        out_shape=jax.ShapeDtypeStruct((B, DIM), jnp.float32),
        compiler_params=pltpu.CompilerParams(
            dimension_semantics=("parallel",)),
    )(m1, m2, m3)


def kernel(story, C0, C1, C2, C3):
    M, B, T_ = story.shape
    S = B * M
    segs = story.transpose(1, 0, 2).reshape(S, T_)
    story_r3 = segs.reshape(S // W, W, T_).transpose(0, 2, 1)
    m1, m2, m3 = _sc_gather_sum(story_r3, C1, C2, C3)
    return _attention(m1.reshape(B, M, DIM),
                      m2.reshape(B, M, DIM),
                      m3.reshape(B, M, DIM))
